# baseline (speedup 1.0000x reference)
import jax
import jax.numpy as jnp
from jax.experimental import pallas as pl
from jax.experimental.pallas import tpu as pltpu

_B = 4096
_D = 128
_TILE = 1024
_NDMA = _B // _TILE


def _bcast(table_ref, out_hbm, tile_v, sem):
    tile_v[...] = jnp.broadcast_to(table_ref[...], (_TILE, _D))
    copies = [
        pltpu.make_async_copy(tile_v, out_hbm.at[pl.ds(k * _TILE, _TILE)], sem)
        for k in range(_NDMA)
    ]
    for c in copies:
        c.start()
    for c in copies:
        c.wait()


@jax.jit
def kernel(x, table):
    del x
    return pl.pallas_call(
        _bcast,
        out_specs=pl.BlockSpec(memory_space=pl.ANY),
        out_shape=jax.ShapeDtypeStruct((_B, _D), jnp.float32),
        scratch_shapes=[
            pltpu.VMEM((_TILE, _D), jnp.float32),
            pltpu.SemaphoreType.DMA,
        ],
    )(table)


# speedup vs baseline: 1.0308x; 1.0308x over previous
import jax
import jax.numpy as jnp
from jax.experimental import pallas as pl
from jax.experimental.pallas import tpu as pltpu

_B = 4096
_D = 128
_TILE = 128
_NDMA = _B // _TILE


def _bcast(table_ref, out_hbm, tile_v, sem):
    tile_v[...] = jnp.broadcast_to(table_ref[...], (_TILE, _D))
    copies = [
        pltpu.make_async_copy(tile_v, out_hbm.at[pl.ds(k * _TILE, _TILE)], sem)
        for k in range(_NDMA)
    ]
    for c in copies:
        c.start()
    for c in copies:
        c.wait()


@jax.jit
def kernel(x, table):
    del x
    return pl.pallas_call(
        _bcast,
        out_specs=pl.BlockSpec(memory_space=pl.ANY),
        out_shape=jax.ShapeDtypeStruct((_B, _D), jnp.float32),
        scratch_shapes=[
            pltpu.VMEM((_TILE, _D), jnp.float32),
            pltpu.SemaphoreType.DMA,
        ],
    )(table)
